# baseline (speedup 1.0000x reference)
import functools

import jax
import jax.numpy as jnp
from jax.experimental import pallas as pl
from jax.experimental.pallas import tpu as pltpu

N_T = 50000
N_U = 10000
N_M = 5000
D = 128
E = 200000

_BLK = 512


def _head_body(ag_ut, ag_mt, ag_tt, id_ut, id_mt, id_tt,
               w_ut, w_mt, w_tt, bsum, w_out, b_out,
               emb_ref, logits_ref):
    rs = lambda d: jax.lax.rsqrt(jnp.maximum(d, 1.0))
    e = jnp.dot(ag_ut[...] * rs(id_ut[...]), w_ut[...],
                preferred_element_type=jnp.float32)
    e += jnp.dot(ag_mt[...] * rs(id_mt[...]), w_mt[...],
                 preferred_element_type=jnp.float32)
    e += jnp.dot(ag_tt[...] * rs(id_tt[...]), w_tt[...],
                 preferred_element_type=jnp.float32)
    e += bsum[...]
    emb_ref[...] = e
    logits_ref[...] = jnp.dot(e, w_out[...],
                              preferred_element_type=jnp.float32) + b_out[...]


def _head(ag_ut, ag_mt, ag_tt, in_ut, in_mt, in_tt,
          W_ut, W_mt, W_tt, bsum, W_out, b_out):
    n_blk = pl.cdiv(N_T, _BLK)
    row = pl.BlockSpec((_BLK, D), lambda i: (i, 0))
    col = pl.BlockSpec((_BLK, 1), lambda i: (i, 0))
    full = pl.BlockSpec((D, D), lambda i: (0, 0))
    return pl.pallas_call(
        _head_body,
        grid=(n_blk,),
        in_specs=[row, row, row, col, col, col, full, full, full,
                  pl.BlockSpec((1, D), lambda i: (0, 0)),
                  pl.BlockSpec((D, 1), lambda i: (0, 0)),
                  pl.BlockSpec((1, 1), lambda i: (0, 0))],
        out_specs=[row, col],
        out_shape=[jax.ShapeDtypeStruct((N_T, D), jnp.float32),
                   jax.ShapeDtypeStruct((N_T, 1), jnp.float32)],
    )(ag_ut, ag_mt, ag_tt, in_ut, in_mt, in_tt,
      W_ut, W_mt, W_tt, bsum, W_out, b_out)


def _agg(h_src, src, dst, n_src):
    ones = jnp.ones(src.shape[0], dtype=jnp.float32)
    out_deg = jnp.maximum(jax.ops.segment_sum(ones, src, num_segments=n_src), 1.0)
    in_deg = jax.ops.segment_sum(ones, dst, num_segments=N_T)
    h_norm = h_src * jax.lax.rsqrt(out_deg)[:, None]
    msgs = jnp.take(h_norm, src, axis=0)
    agg = jax.ops.segment_sum(msgs, dst, num_segments=N_T)
    return agg, in_deg


def kernel(features, user_ids, merchant_ids,
           src_ut, dst_ut, src_mt, dst_mt, src_tt, dst_tt,
           emb_user, emb_merchant,
           W_ut, b_ut, W_mt, b_mt, W_tt, b_tt, W_out, b_out):
    h_user = jnp.take(emb_user, user_ids, axis=0)
    h_merchant = jnp.take(emb_merchant, merchant_ids, axis=0)

    ag_ut, in_ut = _agg(h_user, src_ut, dst_ut, N_U)
    ag_mt, in_mt = _agg(h_merchant, src_mt, dst_mt, N_M)
    ag_tt, in_tt = _agg(features, src_tt, dst_tt, N_T)

    bsum = (b_ut + b_mt + b_tt).reshape(1, D)
    emb, logits = _head(ag_ut, ag_mt, ag_tt,
                        in_ut.reshape(N_T, 1), in_mt.reshape(N_T, 1),
                        in_tt.reshape(N_T, 1),
                        W_ut, W_mt, W_tt, bsum, W_out, b_out.reshape(1, 1))
    return (logits, emb)
